# baseline (speedup 1.0000x reference)
import functools

import jax
import jax.numpy as jnp
from jax import lax
from jax.experimental import pallas as pl
from jax.experimental.pallas import tpu as pltpu
from jax.experimental.pallas import tpu_sc as plsc

N = 10000
D = 128
E = 320000
NC = 2
NS = 16
NW = NC * NS
EPT = 10240
EPAD = NW * EPT
CHUNK = 80
NCH = EPT // CHUNK
NPAD = 10240
RPT = NPAD // NS

NBUF = 4
NIB = 8

BLK = 256
NBLK = NPAD // BLK


def _deg_body(dst_hbm, zeros_hbm, out_hbm, hist, idxf):
    c = lax.axis_index("c")
    s = lax.axis_index("s")
    wid = s * NC + c
    pltpu.sync_copy(zeros_hbm, hist)
    pltpu.sync_copy(dst_hbm.at[wid], idxf)

    def it(i, carry):
        for u in range(4):
            v = idxf[pl.ds((i * 4 + u) * 16, 16)]
            cnt, last = plsc.scan_count(v)
            plsc.addupdate_scatter(hist, [v], cnt, mask=last)
        return carry

    lax.fori_loop(0, EPT // 64, it, 0)
    pltpu.sync_copy(hist, out_hbm.at[wid])


@functools.cache
def _sc_mesh():
    return plsc.VectorSubcoreMesh(
        core_axis_name="c", subcore_axis_name="s",
        num_cores=NC, num_subcores=NS)


@functools.cache
def _deg_call():
    return pl.kernel(
        _deg_body,
        out_type=jax.ShapeDtypeStruct((NW, NPAD), jnp.int32),
        mesh=_sc_mesh(),
        scratch_types=[
            pltpu.VMEM((NPAD,), jnp.int32),
            pltpu.VMEM((EPT,), jnp.int32),
        ],
        compiler_params=pltpu.CompilerParams(needs_layout_passes=False),
    )


def _agg_body(y_hbm, src_hbm, dst_hbm, zeros_hbm, out_hbm,
              acc, sib, dib, rb0, rb1, rb2, rb3,
              i0, i1, i2, i3, i4, i5, i6, i7, g0, g1, g2, g3, t0, t1, t2, t3):
    rbs = (rb0, rb1, rb2, rb3)
    isem = (i0, i1, i2, i3, i4, i5, i6, i7)
    gsem = (g0, g1, g2, g3)
    ssem = (t0, t1, t2, t3)
    c = lax.axis_index("c")
    s = lax.axis_index("s")
    wid = s * NC + c
    rows = pl.ds(s * RPT, RPT)

    def fetch_idx(j, k):
        pltpu.async_copy(src_hbm.at[wid, j], sib.at[k], isem[k])
        pltpu.async_copy(dst_hbm.at[wid, j], dib.at[k], isem[k])

    def wait_idx(j, k):
        pltpu.make_async_copy(src_hbm.at[wid, j], sib.at[k], isem[k]).wait()
        pltpu.make_async_copy(dst_hbm.at[wid, j], dib.at[k], isem[k]).wait()

    def gather(k, b):
        pltpu.async_copy(y_hbm.at[sib.at[k]], rbs[b], gsem[b])

    def scatter(k, b):
        pltpu.async_copy(rbs[b], acc.at[dib.at[k]], ssem[b], add=True)

    def wait_gather(b):
        pltpu.make_async_copy(y_hbm.at[sib.at[0]], rbs[b], gsem[b]).wait()

    def wait_scatter(b):
        pltpu.make_async_copy(rbs[b], acc.at[dib.at[0]], ssem[b]).wait()

    for k in range(4):
        fetch_idx(k, k)
    pltpu.sync_copy(zeros_hbm.at[rows], acc.at[rows])
    wait_idx(0, 0)
    gather(0, 0)
    wait_idx(1, 1)
    gather(1, 1)
    plsc.subcore_barrier()

    def body(i, carry):
        j0 = i * NIB
        for u in range(NIB):
            j = j0 + u

            @pl.when(j + 4 < NCH)
            def _():
                fetch_idx(j + 4, (u + 4) % NIB)

            @pl.when(j >= 2)
            def _():
                wait_scatter((u + 2) % NBUF)

            @pl.when(j + 2 < NCH)
            def _():
                wait_idx(j + 2, (u + 2) % NIB)
                gather((u + 2) % NIB, (u + 2) % NBUF)

            wait_gather(u % NBUF)
            scatter(u % NIB, u % NBUF)
        return carry

    lax.fori_loop(0, NCH // NIB, body, 0)
    wait_scatter((NCH - 2) % NBUF)
    wait_scatter((NCH - 1) % NBUF)
    plsc.subcore_barrier()
    pltpu.sync_copy(acc.at[rows], out_hbm.at[c, rows])


@functools.cache
def _agg_call():
    return pl.kernel(
        _agg_body,
        out_type=jax.ShapeDtypeStruct((NC, NPAD, D), jnp.float32),
        mesh=_sc_mesh(),
        scratch_types=[
            pltpu.VMEM_SHARED((NPAD, D), jnp.float32),
            pltpu.VMEM((NIB, CHUNK), jnp.int32),
            pltpu.VMEM((NIB, CHUNK), jnp.int32),
        ] + [pltpu.VMEM((CHUNK, D), jnp.float32)] * NBUF
          + [pltpu.SemaphoreType.DMA] * (NIB + 2 * NBUF),
    )


def _dinv_of(degp):
    return lax.rsqrt(1.0 + jnp.sum(degp, axis=0).astype(jnp.float32))


def _k0_body(x_ref, w_ref, xw_ref):
    xw = jnp.dot(x_ref[...], w_ref[...], preferred_element_type=jnp.float32)
    row = pl.program_id(0) * BLK + lax.broadcasted_iota(jnp.int32, (BLK, 1), 0)
    xw_ref[...] = jnp.where(row < N, xw, 0.0)


def _k1_body(xw_ref, degp_ref, y_ref):
    dinv = _dinv_of(degp_ref[...])
    y_ref[...] = xw_ref[...] * dinv[:, None]


def _k2_body(s_ref, y1_ref, degp_ref, w_ref, b_ref, y2_ref):
    dinv = _dinv_of(degp_ref[...])
    agg = s_ref[0] + s_ref[1] + y1_ref[...]
    h = jnp.maximum(agg * dinv[:, None] + b_ref[...], 0.0)
    y2 = jnp.dot(h, w_ref[...], preferred_element_type=jnp.float32) * dinv[:, None]
    row = pl.program_id(0) * BLK + lax.broadcasted_iota(jnp.int32, (BLK, 1), 0)
    y2_ref[...] = jnp.where(row < N, y2, 0.0)


def _k3_body(s_ref, y2_ref, degp_ref, b_ref, o_ref):
    dinv = _dinv_of(degp_ref[...])
    o_ref[...] = (s_ref[0] + s_ref[1] + y2_ref[...]) * dinv[:, None] + b_ref[...]


_spec_rows = pl.BlockSpec((BLK, D), lambda i: (i, 0))
_spec_w = pl.BlockSpec((D, D), lambda i: (0, 0))
_spec_degp = pl.BlockSpec((NW, BLK), lambda i: (0, i))
_spec_s = pl.BlockSpec((NC, BLK, D), lambda i: (0, i, 0))
_spec_b = pl.BlockSpec((1, D), lambda i: (0, 0))

_k0 = pl.pallas_call(
    _k0_body,
    grid=(NBLK,),
    in_specs=[_spec_rows, _spec_w],
    out_specs=_spec_rows,
    out_shape=jax.ShapeDtypeStruct((NPAD, D), jnp.float32),
)

_k1 = pl.pallas_call(
    _k1_body,
    grid=(NBLK,),
    in_specs=[_spec_rows, _spec_degp],
    out_specs=_spec_rows,
    out_shape=jax.ShapeDtypeStruct((NPAD, D), jnp.float32),
)

_k2 = pl.pallas_call(
    _k2_body,
    grid=(NBLK,),
    in_specs=[_spec_s, _spec_rows, _spec_degp, _spec_w, _spec_b],
    out_specs=_spec_rows,
    out_shape=jax.ShapeDtypeStruct((NPAD, D), jnp.float32),
)

_k3 = pl.pallas_call(
    _k3_body,
    grid=(NBLK,),
    in_specs=[_spec_s, _spec_rows, _spec_degp, _spec_b],
    out_specs=_spec_rows,
    out_shape=jax.ShapeDtypeStruct((NPAD, D), jnp.float32),
)


def kernel(x, edge_index, W1, b1, W2, b2):
    src = edge_index[0].astype(jnp.int32)
    dst = edge_index[1].astype(jnp.int32)
    pad = (jnp.arange(EPAD - E, dtype=jnp.int32) % (NPAD - N)) + N
    src_p = jnp.concatenate([src, pad]).reshape(NW, NCH, CHUNK)
    dst_p = jnp.concatenate([dst, pad]).reshape(NW, NCH, CHUNK)
    dst_flat = dst_p.reshape(NW, EPT)
    zeros_d = jnp.zeros((NPAD, D), jnp.float32)
    zeros_1 = jnp.zeros((NPAD,), jnp.int32)
    b1r = b1.reshape(1, D)
    b2r = b2.reshape(1, D)

    degp = _deg_call()(dst_flat, zeros_1)
    xw1 = _k0(x, W1)
    y1 = _k1(xw1, degp)
    s1 = _agg_call()(y1, src_p, dst_p, zeros_d)
    y2 = _k2(s1, y1, degp, W2, b1r)
    s2 = _agg_call()(y2, src_p, dst_p, zeros_d)
    out = _k3(s2, y2, degp, b2r)
    return out[:N]


# speedup vs baseline: 1.0358x; 1.0358x over previous
import functools

import jax
import jax.numpy as jnp
from jax import lax
from jax.experimental import pallas as pl
from jax.experimental.pallas import tpu as pltpu
from jax.experimental.pallas import tpu_sc as plsc

N = 10000
D = 128
E = 320000
NC = 2
NS = 16
NW = NC * NS
EPT = 10240
EPAD = NW * EPT
CHUNK = 80
NCH = EPT // CHUNK
NPAD = 10240
RPT = NPAD // NS

NBUF = 4
NIB = 8

BLK = 256
NBLK = NPAD // BLK


def _deg_body(dst_hbm, zeros_hbm, out_hbm, hist, idxf):
    c = lax.axis_index("c")
    s = lax.axis_index("s")
    wid = s * NC + c
    pltpu.sync_copy(zeros_hbm, hist)
    pltpu.sync_copy(dst_hbm.at[wid], idxf)

    def it(i, carry):
        for u in range(4):
            v = idxf[pl.ds((i * 4 + u) * 16, 16)]
            cnt, last = plsc.scan_count(v)
            plsc.addupdate_scatter(hist, [v], cnt, mask=last)
        return carry

    lax.fori_loop(0, EPT // 64, it, 0)
    pltpu.sync_copy(hist, out_hbm.at[wid])


@functools.cache
def _sc_mesh():
    return plsc.VectorSubcoreMesh(
        core_axis_name="c", subcore_axis_name="s",
        num_cores=NC, num_subcores=NS)


@functools.cache
def _deg_call():
    return pl.kernel(
        _deg_body,
        out_type=jax.ShapeDtypeStruct((NW, NPAD), jnp.int32),
        mesh=_sc_mesh(),
        scratch_types=[
            pltpu.VMEM((NPAD,), jnp.int32),
            pltpu.VMEM((EPT,), jnp.int32),
        ],
        compiler_params=pltpu.CompilerParams(needs_layout_passes=False),
    )


def _agg_body(y_hbm, src_hbm, dst_hbm, zeros_hbm, out_hbm,
              acc, sib, dib, rb0, rb1, rb2, rb3,
              i0, i1, i2, i3, i4, i5, i6, i7, g0, g1, g2, g3, t0, t1, t2, t3):
    rbs = (rb0, rb1, rb2, rb3)
    isem = (i0, i1, i2, i3, i4, i5, i6, i7)
    gsem = (g0, g1, g2, g3)
    ssem = (t0, t1, t2, t3)
    c = lax.axis_index("c")
    s = lax.axis_index("s")
    wid = s * NC + c
    rows = pl.ds(s * RPT, RPT)

    def fetch_idx(j, k):
        pltpu.async_copy(src_hbm.at[wid, j], sib.at[k], isem[k])
        pltpu.async_copy(dst_hbm.at[wid, j], dib.at[k], isem[k])

    def wait_idx(j, k):
        pltpu.make_async_copy(src_hbm.at[wid, j], sib.at[k], isem[k]).wait()
        pltpu.make_async_copy(dst_hbm.at[wid, j], dib.at[k], isem[k]).wait()

    def gather(k, b):
        pltpu.async_copy(y_hbm.at[sib.at[k]], rbs[b], gsem[b])

    def scatter(k, b):
        pltpu.async_copy(rbs[b], acc.at[dib.at[k]], ssem[b], add=True)

    def wait_gather(b):
        pltpu.make_async_copy(y_hbm.at[sib.at[0]], rbs[b], gsem[b]).wait()

    def wait_scatter(b):
        pltpu.make_async_copy(rbs[b], acc.at[dib.at[0]], ssem[b]).wait()

    for k in range(4):
        fetch_idx(k, k)
    pltpu.sync_copy(zeros_hbm.at[rows], acc.at[rows])
    wait_idx(0, 0)
    gather(0, 0)
    wait_idx(1, 1)
    gather(1, 1)
    plsc.subcore_barrier()

    def body(i, carry):
        j0 = i * NIB
        for u in range(NIB):
            j = j0 + u

            @pl.when(j + 4 < NCH)
            def _():
                fetch_idx(j + 4, (u + 4) % NIB)

            @pl.when(j >= 2)
            def _():
                wait_scatter((u + 2) % NBUF)

            @pl.when(j + 2 < NCH)
            def _():
                wait_idx(j + 2, (u + 2) % NIB)
                gather((u + 2) % NIB, (u + 2) % NBUF)

            wait_gather(u % NBUF)
            scatter(u % NIB, u % NBUF)
        return carry

    lax.fori_loop(0, NCH // NIB, body, 0)
    wait_scatter((NCH - 2) % NBUF)
    wait_scatter((NCH - 1) % NBUF)
    plsc.subcore_barrier()
    pltpu.sync_copy(acc.at[rows], out_hbm.at[c, rows])


@functools.cache
def _agg_call():
    return pl.kernel(
        _agg_body,
        out_type=jax.ShapeDtypeStruct((NC, NPAD, D), jnp.float32),
        mesh=_sc_mesh(),
        scratch_types=[
            pltpu.VMEM_SHARED((NPAD, D), jnp.float32),
            pltpu.VMEM((NIB, CHUNK), jnp.int32),
            pltpu.VMEM((NIB, CHUNK), jnp.int32),
        ] + [pltpu.VMEM((CHUNK, D), jnp.float32)] * NBUF
          + [pltpu.SemaphoreType.DMA] * (NIB + 2 * NBUF),
    )


def _dinv_of(degp):
    return lax.rsqrt(1.0 + jnp.sum(degp, axis=0).astype(jnp.float32))


def _k1_body(x_ref, w_ref, degp_ref, y_ref):
    dinv = _dinv_of(degp_ref[...])
    xw = jnp.dot(x_ref[...], w_ref[...], preferred_element_type=jnp.float32)
    row = pl.program_id(0) * BLK + lax.broadcasted_iota(jnp.int32, (BLK, 1), 0)
    y_ref[...] = jnp.where(row < N, xw * dinv[:, None], 0.0)


def _k2_body(s_ref, y1_ref, degp_ref, w_ref, b_ref, y2_ref):
    dinv = _dinv_of(degp_ref[...])
    agg = s_ref[0] + s_ref[1] + y1_ref[...]
    h = jnp.maximum(agg * dinv[:, None] + b_ref[...], 0.0)
    y2 = jnp.dot(h, w_ref[...], preferred_element_type=jnp.float32) * dinv[:, None]
    row = pl.program_id(0) * BLK + lax.broadcasted_iota(jnp.int32, (BLK, 1), 0)
    y2_ref[...] = jnp.where(row < N, y2, 0.0)


def _k3_body(s_ref, y2_ref, degp_ref, b_ref, o_ref):
    dinv = _dinv_of(degp_ref[...])
    o_ref[...] = (s_ref[0] + s_ref[1] + y2_ref[...]) * dinv[:, None] + b_ref[...]


_spec_rows = pl.BlockSpec((BLK, D), lambda i: (i, 0))
_spec_w = pl.BlockSpec((D, D), lambda i: (0, 0))
_spec_degp = pl.BlockSpec((NW, BLK), lambda i: (0, i))
_spec_s = pl.BlockSpec((NC, BLK, D), lambda i: (0, i, 0))
_spec_b = pl.BlockSpec((1, D), lambda i: (0, 0))

_k1 = pl.pallas_call(
    _k1_body,
    grid=(NBLK,),
    in_specs=[_spec_rows, _spec_w, _spec_degp],
    out_specs=_spec_rows,
    out_shape=jax.ShapeDtypeStruct((NPAD, D), jnp.float32),
)

_k2 = pl.pallas_call(
    _k2_body,
    grid=(NBLK,),
    in_specs=[_spec_s, _spec_rows, _spec_degp, _spec_w, _spec_b],
    out_specs=_spec_rows,
    out_shape=jax.ShapeDtypeStruct((NPAD, D), jnp.float32),
)

_k3 = pl.pallas_call(
    _k3_body,
    grid=(NBLK,),
    in_specs=[_spec_s, _spec_rows, _spec_degp, _spec_b],
    out_specs=_spec_rows,
    out_shape=jax.ShapeDtypeStruct((NPAD, D), jnp.float32),
)


def kernel(x, edge_index, W1, b1, W2, b2):
    src = edge_index[0].astype(jnp.int32)
    dst = edge_index[1].astype(jnp.int32)
    pad = (jnp.arange(EPAD - E, dtype=jnp.int32) % (NPAD - N)) + N
    src_p = jnp.concatenate([src, pad]).reshape(NW, NCH, CHUNK)
    dst_p = jnp.concatenate([dst, pad]).reshape(NW, NCH, CHUNK)
    dst_flat = dst_p.reshape(NW, EPT)
    zeros_d = jnp.zeros((NPAD, D), jnp.float32)
    zeros_1 = jnp.zeros((NPAD,), jnp.int32)
    b1r = b1.reshape(1, D)
    b2r = b2.reshape(1, D)

    degp = _deg_call()(dst_flat, zeros_1, )
    y1 = _k1(x, W1, degp)
    s1 = _agg_call()(y1, src_p, dst_p, zeros_d)
    y2 = _k2(s1, y1, degp, W2, b1r)
    s2 = _agg_call()(y2, src_p, dst_p, zeros_d)
    out = _k3(s2, y2, degp, b2r)
    return out[:N]
